# Initial kernel scaffold; baseline (speedup 1.0000x reference)
import jax
import jax.numpy as jnp
from jax.experimental import pallas as pl


def kernel(x, gate_w, Wg, Wu, Wd):
    raise NotImplementedError("write your pallas kernel here")


# speedup vs baseline: 3.8054x; 3.8054x over previous
import functools

import jax
import jax.numpy as jnp
from jax import lax
from jax.experimental import pallas as pl
from jax.experimental.pallas import tpu as pltpu
from jax.experimental.pallas import tpu_sc as plsc

E = 8
TOP_K = 2
DIM = 2048
DFF = 8192
B = 512
C = 512
TB = 512


def _router_body(x_ref, gw_ref, idx_ref, w_ref):
    s = lax.dot_general(x_ref[...], gw_ref[...], (((1,), (1,)), ((), ())),
                        preferred_element_type=jnp.float32)
    cols = lax.broadcasted_iota(jnp.int32, s.shape, 1)
    m1 = jnp.max(s, axis=1, keepdims=True)
    a1 = jnp.min(jnp.where(s == m1, cols, E), axis=1, keepdims=True)
    s2 = jnp.where(cols == a1, -jnp.inf, s)
    m2 = jnp.max(s2, axis=1, keepdims=True)
    a2 = jnp.min(jnp.where(s2 == m2, cols, E), axis=1, keepdims=True)
    ew = jnp.exp(m2 - m1)
    w1 = 1.0 / (1.0 + ew)
    k2 = lax.broadcasted_iota(jnp.int32, (a1.shape[0], 2), 1)
    idx_ref[...] = jnp.where(k2 == 0, a1, a2)
    w_ref[...] = jnp.where(k2 == 0, w1, 1.0 - w1)


def _router(xf, gate_w):
    n = xf.shape[0]
    return pl.pallas_call(
        _router_body,
        grid=(n // TB,),
        in_specs=[
            pl.BlockSpec((TB, DIM), lambda i: (i, 0)),
            pl.BlockSpec((E, DIM), lambda i: (0, 0)),
        ],
        out_specs=[
            pl.BlockSpec((TB, 2), lambda i: (i, 0)),
            pl.BlockSpec((TB, 2), lambda i: (i, 0)),
        ],
        out_shape=[
            jax.ShapeDtypeStruct((n, 2), jnp.int32),
            jax.ShapeDtypeStruct((n, 2), jnp.float32),
        ],
    )(xf, gate_w)


def _make_sc_gather(n_out, d, chunk):
    info = plsc.get_sparse_core_info()
    nw = info.num_cores * info.num_subcores
    per_w = n_out // nw
    nchunks = per_w // chunk
    assert per_w % chunk == 0 and n_out % nw == 0
    mesh = plsc.VectorSubcoreMesh(core_axis_name="c", subcore_axis_name="s")

    def body(table_hbm, idx_hbm, out_hbm, idx_v, rows_v, sem):
        wid = lax.axis_index("s") * info.num_cores + lax.axis_index("c")
        base = wid * per_w
        pltpu.sync_copy(idx_hbm.at[pl.ds(base, per_w)], idx_v)
        for k in range(nchunks):
            pltpu.async_copy(
                table_hbm.at[idx_v.at[pl.ds(k * chunk, chunk)]], rows_v, sem
            ).wait()
            pltpu.sync_copy(rows_v, out_hbm.at[pl.ds(base + k * chunk, chunk)])

    return pl.kernel(
        body,
        out_type=jax.ShapeDtypeStruct((n_out, d), jnp.float32),
        mesh=mesh,
        scratch_types=[
            pltpu.VMEM((per_w,), jnp.int32),
            pltpu.VMEM((chunk, d), jnp.float32),
            pltpu.SemaphoreType.DMA,
        ],
    )


def _ffn_body(be_ref, valid_ref, x_ref, wg_ref, wu_ref, wd_ref, ws_ref, y_ref,
              acc_ref, *, nj):
    i = pl.program_id(0)
    j = pl.program_id(1)

    @pl.when(j == 0)
    def _init():
        acc_ref[...] = jnp.zeros_like(acc_ref)

    @pl.when(valid_ref[i] > 0)
    def _compute():
        x = x_ref[...]
        g = lax.dot_general(x, wg_ref[0], (((1,), (1,)), ((), ())),
                            preferred_element_type=jnp.float32)
        u = lax.dot_general(x, wu_ref[0], (((1,), (1,)), ((), ())),
                            preferred_element_type=jnp.float32)
        h = g * lax.logistic(g) * u
        acc_ref[...] += lax.dot_general(h, wd_ref[0], (((1,), (1,)), ((), ())),
                                        preferred_element_type=jnp.float32)

    @pl.when(j == nj - 1)
    def _store():
        y_ref[...] = acc_ref[...] * ws_ref[...]


def _ffn(be, valid, x_pad, Wg, Wu, Wd, w_slot):
    p = x_pad.shape[0]
    g_blocks = p // B
    nj = DFF // C
    grid_spec = pltpu.PrefetchScalarGridSpec(
        num_scalar_prefetch=2,
        grid=(g_blocks, nj),
        in_specs=[
            pl.BlockSpec((B, DIM), lambda i, j, be, va: (i, 0)),
            pl.BlockSpec((1, C, DIM), lambda i, j, be, va: (be[i], j, 0)),
            pl.BlockSpec((1, C, DIM), lambda i, j, be, va: (be[i], j, 0)),
            pl.BlockSpec((1, DIM, C), lambda i, j, be, va: (be[i], 0, j)),
            pl.BlockSpec((B, 1), lambda i, j, be, va: (i, 0)),
        ],
        out_specs=pl.BlockSpec((B, DIM), lambda i, j, be, va: (i, 0)),
        scratch_shapes=[pltpu.VMEM((B, DIM), jnp.float32)],
    )
    return pl.pallas_call(
        functools.partial(_ffn_body, nj=nj),
        grid_spec=grid_spec,
        out_shape=jax.ShapeDtypeStruct((p, DIM), jnp.float32),
        compiler_params=pltpu.CompilerParams(
            dimension_semantics=("arbitrary", "arbitrary")),
    )(be, valid, x_pad, Wg, Wu, Wd, w_slot[:, None])


def _combine_body(y_ref, o_ref):
    o_ref[...] = y_ref[:, 0, :] + y_ref[:, 1, :]


def _combine(y_pair):
    n = y_pair.shape[0] // 2
    return pl.pallas_call(
        _combine_body,
        grid=(n // TB,),
        in_specs=[pl.BlockSpec((TB, 2, DIM), lambda i: (i, 0, 0))],
        out_specs=pl.BlockSpec((TB, DIM), lambda i: (i, 0)),
        out_shape=jax.ShapeDtypeStruct((n, DIM), jnp.float32),
    )(y_pair.reshape(n, 2, DIM))


def kernel(x, gate_w, Wg, Wu, Wd):
    orig_shape = x.shape
    xf = x.reshape(-1, x.shape[-1])
    n = xf.shape[0]
    n2 = n * TOP_K
    g_blocks = n2 // B + E
    p = g_blocks * B

    idx, wts = _router(xf, gate_w)

    fe = idx.reshape(-1)
    order = jnp.argsort(fe, stable=True).astype(jnp.int32)
    counts = jnp.zeros((E,), jnp.int32).at[fe].add(1)
    nb = (counts + B - 1) // B
    bcum = jnp.cumsum(nb)
    astart = (jnp.concatenate([jnp.zeros((1,), bcum.dtype), bcum[:-1]]) * B)
    gstart = jnp.concatenate(
        [jnp.zeros((1,), counts.dtype), jnp.cumsum(counts)[:-1]])
    e_sorted = fe[order]
    m = jnp.arange(n2, dtype=jnp.int32)
    pos_sorted = (astart[e_sorted] + (m - gstart[e_sorted])).astype(jnp.int32)
    tok_slot = jnp.zeros((p,), jnp.int32).at[pos_sorted].set(order // TOP_K)
    w_slot = jnp.zeros((p,), jnp.float32).at[pos_sorted].set(
        wts.reshape(-1)[order])
    pos_flat = jnp.zeros((n2,), jnp.int32).at[order].set(pos_sorted)
    blk = jnp.arange(g_blocks, dtype=bcum.dtype)
    block_expert = jnp.clip(
        jnp.searchsorted(bcum, blk, side="right"), 0, E - 1).astype(jnp.int32)
    valid = (blk < bcum[E - 1]).astype(jnp.int32)

    x_pad = _make_sc_gather(p, DIM, 32)(xf, tok_slot)
    y_pad = _ffn(block_expert, valid, x_pad, Wg, Wu, Wd, w_slot)
    y_pair = _make_sc_gather(n2, DIM, 32)(y_pad, pos_flat)
    out = _combine(y_pair)
    return out.reshape(orig_shape)
